# baseline (speedup 1.0000x reference)
import jax
import jax.numpy as jnp
from jax import lax
from jax.experimental import pallas as pl
from jax.experimental.pallas import tpu as pltpu
from jax.experimental.pallas import tpu_sc as plsc

NUM_UNITS = 32
MAX_REL = 128
LQ = 2048
LK = 2048
E_ROWS = LQ + LK - 1
BAND_LO = LK - 1 - MAX_REL
BAND_ROWS = 2 * MAX_REL + 1
NW = 32
ROWS_PER_W = LQ // NW
ROW_W = LK * NUM_UNITS


def _sc_body(table_hbm, out_hbm, e_vmem, sem):
    c = lax.axis_index("c")
    s = lax.axis_index("s")
    wid = s * 2 + c

    pltpu.sync_copy(table_hbm, e_vmem.at[pl.ds(BAND_LO * NUM_UNITS, BAND_ROWS * NUM_UNITS)])

    t0a = e_vmem[pl.ds(BAND_LO * NUM_UNITS, 16)]
    t0b = e_vmem[pl.ds(BAND_LO * NUM_UNITS + 16, 16)]
    t1a = e_vmem[pl.ds((BAND_LO + BAND_ROWS - 1) * NUM_UNITS, 16)]
    t1b = e_vmem[pl.ds((BAND_LO + BAND_ROWS - 1) * NUM_UNITS + 16, 16)]

    def fill_head(k, _):
        e_vmem[pl.ds(k * NUM_UNITS, 16)] = t0a
        e_vmem[pl.ds(k * NUM_UNITS + 16, 16)] = t0b
        return 0

    lax.fori_loop(0, BAND_LO, fill_head, 0)

    def fill_tail(k, _):
        e_vmem[pl.ds(k * NUM_UNITS, 16)] = t1a
        e_vmem[pl.ds(k * NUM_UNITS + 16, 16)] = t1b
        return 0

    lax.fori_loop(BAND_LO + BAND_ROWS, E_ROWS, fill_tail, 0)

    def row_step(r, _):
        i = wid * ROWS_PER_W + r
        pltpu.async_copy(e_vmem.at[pl.ds((LK - 1 - i) * NUM_UNITS, ROW_W)],
                         out_hbm.at[pl.ds(i * ROW_W, ROW_W)], sem)
        return 0

    lax.fori_loop(0, ROWS_PER_W, row_step, 0)

    def drain(r, _):
        pltpu.make_async_copy(e_vmem.at[pl.ds(0, ROW_W)],
                              out_hbm.at[pl.ds(0, ROW_W)], sem).wait()
        return 0

    lax.fori_loop(0, ROWS_PER_W, drain, 0)


def kernel(x, embeddings_table):
    del x
    run = pl.kernel(
        _sc_body,
        out_type=jax.ShapeDtypeStruct((LQ * ROW_W,), jnp.float32),
        mesh=plsc.VectorSubcoreMesh(core_axis_name="c", subcore_axis_name="s"),
        scratch_types=[
            pltpu.VMEM((E_ROWS * NUM_UNITS,), jnp.float32),
            pltpu.SemaphoreType.DMA,
        ],
    )
    out = run(embeddings_table.reshape(-1))
    return out.reshape(LQ, LK, NUM_UNITS)


# speedup vs baseline: 9.2212x; 9.2212x over previous
import jax
import jax.numpy as jnp
from jax import lax
from jax.experimental import pallas as pl
from jax.experimental.pallas import tpu as pltpu
from jax.experimental.pallas import tpu_sc as plsc

NUM_UNITS = 32
MAX_REL = 128
LQ = 2048
LK = 2048
NW = 32
ROWS_PER_W = LQ // NW
W_COLS = 2560
UT = NUM_UNITS // 8
JT = LK // 128
TAB_WORDS = (2 * MAX_REL + 1) * NUM_UNITS


def _sc_body(tab_hbm, out_hbm, tab_vmem, w_vmem, sem):
    c = lax.axis_index("c")
    s = lax.axis_index("s")
    wid = s * 2 + c
    phi = wid % 8
    q = wid // 8
    wof = (LK - 1 - 504) - phi - 512 * q

    pltpu.sync_copy(tab_hbm, tab_vmem)

    lanes = lax.iota(jnp.int32, 16)

    def build_u(u, _):
        def build_m(mb, _):
            m = mb * 16
            k = wof + m - (LK - 1) + lanes
            cidx = jnp.clip(k, -MAX_REL, MAX_REL) + MAX_REL
            w_vmem[u, pl.ds(m, 16)] = plsc.load_gather(
                tab_vmem, [cidx * NUM_UNITS + u])
            return 0
        lax.fori_loop(0, W_COLS // 16, build_m, 0)
        return 0

    lax.fori_loop(0, NUM_UNITS, build_u, 0)

    def drain_row():
        for _ in range(UT * JT):
            pltpu.make_async_copy(
                w_vmem.at[pl.ds(0, 8), pl.ds(0, 128)],
                out_hbm.at[0, 0, 0], sem).wait()

    def row(t, _):
        @pl.when(t > 0)
        def _():
            drain_row()
        i = phi + 512 * q + 8 * t
        s0 = 8 * (ROWS_PER_W - 1 - t)
        for ut in range(UT):
            for jt in range(JT):
                pltpu.async_copy(
                    w_vmem.at[pl.ds(8 * ut, 8), pl.ds(s0 + 128 * jt, 128)],
                    out_hbm.at[i, ut, jt], sem)
        return 0

    lax.fori_loop(0, ROWS_PER_W, row, 0)
    drain_row()


def kernel(x, embeddings_table):
    del x
    run = pl.kernel(
        _sc_body,
        out_type=jax.ShapeDtypeStruct((LQ, UT, JT, 8, 128), jnp.float32),
        mesh=plsc.VectorSubcoreMesh(core_axis_name="c", subcore_axis_name="s"),
        scratch_types=[
            pltpu.VMEM((TAB_WORDS,), jnp.float32),
            pltpu.VMEM((NUM_UNITS, W_COLS), jnp.float32),
            pltpu.SemaphoreType.DMA,
        ],
        compiler_params=pltpu.CompilerParams(use_tc_tiling_on_sc=False,
                                             needs_layout_passes=False),
    )
    out5 = run(embeddings_table.reshape(-1))
    return out5.transpose(0, 2, 4, 1, 3).reshape(LQ, LK, NUM_UNITS)


# speedup vs baseline: 9.4329x; 1.0230x over previous
import jax
import jax.numpy as jnp
from jax import lax
from jax.experimental import pallas as pl
from jax.experimental.pallas import tpu as pltpu
from jax.experimental.pallas import tpu_sc as plsc

NUM_UNITS = 32
MAX_REL = 128
LQ = 2048
LK = 2048
NW = 32
ROWS_PER_W = LQ // NW
W_COLS = 2560
UT = NUM_UNITS // 8
JT = LK // 128
TAB_WORDS = (2 * MAX_REL + 1) * NUM_UNITS


def _sc_body(tab_hbm, out_hbm, tab_vmem, w_vmem, sem):
    c = lax.axis_index("c")
    s = lax.axis_index("s")
    wid = s * 2 + c
    phi = wid % 8
    q = wid // 8
    wof = (LK - 1 - 504) - phi - 512 * q

    pltpu.sync_copy(tab_hbm, tab_vmem)

    lanes = lax.iota(jnp.int32, 16)

    def build_u(u, _):
        def build_m(mb, _):
            m = mb * 16
            k = wof + m - (LK - 1) + lanes
            cidx = jnp.clip(k, -MAX_REL, MAX_REL) + MAX_REL
            w_vmem[u // 8, u % 8, pl.ds(m, 16)] = plsc.load_gather(
                tab_vmem, [cidx * NUM_UNITS + u])
            return 0
        lax.fori_loop(0, W_COLS // 16, build_m, 0)
        return 0

    lax.fori_loop(0, NUM_UNITS, build_u, 0)

    def drain_row():
        for _ in range(JT):
            pltpu.make_async_copy(
                w_vmem.at[:, :, pl.ds(0, 128)],
                out_hbm.at[0, :, 0], sem).wait()

    def row(t, _):
        @pl.when(t > 0)
        def _():
            drain_row()
        i = phi + 512 * q + 8 * t
        s0 = 8 * (ROWS_PER_W - 1 - t)
        for jt in range(JT):
            pltpu.async_copy(
                w_vmem.at[:, :, pl.ds(s0 + 128 * jt, 128)],
                out_hbm.at[i, :, jt], sem)
        return 0

    lax.fori_loop(0, ROWS_PER_W, row, 0)
    drain_row()


def kernel(x, embeddings_table):
    del x
    run = pl.kernel(
        _sc_body,
        out_type=jax.ShapeDtypeStruct((LQ, UT, JT, 8, 128), jnp.float32),
        mesh=plsc.VectorSubcoreMesh(core_axis_name="c", subcore_axis_name="s"),
        scratch_types=[
            pltpu.VMEM((TAB_WORDS,), jnp.float32),
            pltpu.VMEM((UT, 8, W_COLS), jnp.float32),
            pltpu.SemaphoreType.DMA,
        ],
        compiler_params=pltpu.CompilerParams(use_tc_tiling_on_sc=False,
                                             needs_layout_passes=False),
    )
    out5 = run(embeddings_table.reshape(-1))
    return out5.transpose(0, 2, 4, 1, 3).reshape(LQ, LK, NUM_UNITS)
